# baseline (speedup 1.0000x reference)
import functools

import jax
import jax.numpy as jnp
from jax import lax
from jax.experimental import pallas as pl
from jax.experimental.pallas import tpu as pltpu
from jax.experimental.pallas import tpu_sc as plsc

B, T, D = 16, 512, 256
L = 2048
LANES = 16
NW = 32
POS_PER_W = (B * L) // NW
CHUNK = 128
NCHUNK = POS_PER_W // CHUNK


def _body(tbl_hbm, dur_hbm, out_hbm, mel_hbm,
          idx_v, dur_v, durall_v, mel8_v, rows_v, zb_v, sem0, sem1):
    cid = lax.axis_index("c")
    sid = lax.axis_index("s")
    wid = sid * 2 + cid
    b = wid // 2
    half = wid % 2

    pltpu.sync_copy(dur_hbm.at[b], dur_v)
    lim_vec = jnp.full((LANES,), L, jnp.int32)
    iota = lax.iota(jnp.int32, LANES)

    def init_body(i, carry):
        idx_v[pl.ds(i * LANES, LANES)] = iota + ((i * LANES) & (CHUNK - 1))
        return carry

    lax.fori_loop(0, L // LANES, init_body, 0)

    def grp_body(g, c):
        dvec = dur_v[pl.ds(g * LANES, LANES)]
        incl = plsc.cumsum(dvec)
        starts = incl - dvec + c
        vals = jnp.full((LANES,), b * T, jnp.int32) + (g * LANES + iota)
        for r in range(7):
            inds = starts + r
            mask = (r < dvec) & (inds < lim_vec)
            plsc.store_scatter(idx_v, [inds], vals, mask=mask)
        return c + incl[LANES - 1]

    total = lax.fori_loop(0, T // LANES, grp_body, jnp.int32(0))

    sems = (sem0, sem1)

    def gstart(i, k):
        off = (2 * i + half) * CHUNK
        return pltpu.async_copy(
            tbl_hbm.at[idx_v.at[pl.ds(off, CHUNK)]], rows_v.at[k], sems[k])

    pend = [gstart(0, 0), gstart(1, 1)]

    @pl.when(sid == 0)
    def _():
        pltpu.sync_copy(dur_hbm, durall_v)
        melvec = jnp.zeros((LANES,), jnp.int32)
        for m in range(8):
            def acc_body(q, acc, m=m):
                return acc + durall_v[cid * 8 + m, pl.ds(q * LANES, LANES)]
            acc = lax.fori_loop(0, T // LANES, acc_body,
                                jnp.zeros((LANES,), jnp.int32))
            melvec = jnp.where(iota == m, jnp.sum(acc), melvec)
        mel8_v[...] = melvec
        pltpu.sync_copy(mel8_v.at[pl.ds(0, 8)], mel_hbm.at[pl.ds(cid * 8, 8)])

    vend = jnp.clip(total, 0, L)
    zf = jnp.zeros((LANES,), jnp.float32)

    @pl.when(vend <= (2 * (NCHUNK - 1) + half) * CHUNK)
    def _():
        def zb_body(j, carry):
            for q in range(D // LANES):
                zb_v[j, pl.ds(q * LANES, LANES)] = zf
            return carry
        lax.fori_loop(0, CHUNK, zb_body, 0)
    for i in range(NCHUNK):
        k = i % 2
        pend[k].wait()
        v_i = jnp.clip(vend - (2 * i + half) * CHUNK, 0, CHUNK)
        dst = out_hbm.at[b, pl.ds((2 * i + half) * CHUNK, CHUNK)]

        @pl.when(v_i == CHUNK)
        def _(k=k, dst=dst):
            pltpu.sync_copy(rows_v.at[k], dst)

        @pl.when(jnp.logical_and(v_i > 0, v_i < CHUNK))
        def _(k=k, dst=dst, v_i=v_i):
            def row_zero(j, carry):
                for q in range(D // LANES):
                    rows_v[k, j, pl.ds(q * LANES, LANES)] = zf
                return carry
            lax.fori_loop(v_i, CHUNK, row_zero, 0)
            pltpu.sync_copy(rows_v.at[k], dst)

        @pl.when(v_i == 0)
        def _(dst=dst):
            pltpu.sync_copy(zb_v, dst)

        if i + 2 < NCHUNK:
            pend[k] = gstart(i + 2, k)


@jax.jit
def _regulate(tbl, duration):
    mesh = plsc.VectorSubcoreMesh(core_axis_name="c", subcore_axis_name="s")
    fn = pl.kernel(
        _body,
        out_type=(jax.ShapeDtypeStruct((B, L, D), jnp.float32),
                  jax.ShapeDtypeStruct((B,), jnp.int32)),
        mesh=mesh,
        compiler_params=pltpu.CompilerParams(needs_layout_passes=False),
        scratch_types=[
            pltpu.VMEM((L,), jnp.int32),
            pltpu.VMEM((T,), jnp.int32),
            pltpu.VMEM((B, T), jnp.int32),
            pltpu.VMEM((LANES,), jnp.int32),
            pltpu.VMEM((2, CHUNK, D), jnp.float32),
            pltpu.VMEM((CHUNK, D), jnp.float32),
            pltpu.SemaphoreType.DMA,
            pltpu.SemaphoreType.DMA,
        ],
    )
    return fn(tbl, duration)


def kernel(x, duration, max_len):
    del max_len
    return _regulate(x.reshape(B * T, D), duration.astype(jnp.int32))


# speedup vs baseline: 1.0673x; 1.0673x over previous
import functools

import jax
import jax.numpy as jnp
from jax import lax
from jax.experimental import pallas as pl
from jax.experimental.pallas import tpu as pltpu
from jax.experimental.pallas import tpu_sc as plsc

B, T, D = 16, 512, 256
L = 2048
LANES = 16
NW = 32
POS_PER_W = (B * L) // NW
CHUNK = 128
NCHUNK = POS_PER_W // CHUNK


def _body(tbl_hbm, dur_hbm, out_hbm, mel_hbm,
          idx_v, dur_v, durall_v, mel8_v, rows_v, zb_v, sem0, sem1):
    cid = lax.axis_index("c")
    sid = lax.axis_index("s")
    wid = sid * 2 + cid
    b = wid // 2
    half = wid % 2
    base = half * POS_PER_W

    pltpu.sync_copy(dur_hbm.at[b], dur_v)
    lim_vec = jnp.full((LANES,), L, jnp.int32)
    iota = lax.iota(jnp.int32, LANES)

    def init_body(i, carry):
        idx_v[pl.ds(base + i * LANES, LANES)] = (
            iota + (b * T + ((i * LANES) & (CHUNK - 1))))
        return carry

    lax.fori_loop(0, POS_PER_W // LANES, init_body, 0)

    def grp_body(g, c):
        dvec = dur_v[pl.ds(g * LANES, LANES)]
        incl = plsc.cumsum(dvec)
        starts = incl - dvec + c
        vals = jnp.full((LANES,), b * T, jnp.int32) + (g * LANES + iota)
        for r in range(7):
            inds = starts + r
            mask = (r < dvec) & (inds < lim_vec)
            plsc.store_scatter(idx_v, [inds], vals, mask=mask)
        return c + incl[LANES - 1]

    total = lax.fori_loop(0, T // LANES, grp_body, jnp.int32(0))

    sems = (sem0, sem1)

    def gstart(i, k):
        off = base + i * CHUNK
        return pltpu.async_copy(
            tbl_hbm.at[idx_v.at[pl.ds(off, CHUNK)]], rows_v.at[k], sems[k])

    pend = [gstart(0, 0), gstart(1, 1)]

    @pl.when(sid == 0)
    def _():
        pltpu.sync_copy(dur_hbm, durall_v)
        melvec = jnp.zeros((LANES,), jnp.int32)
        for m in range(8):
            def acc_body(q, acc, m=m):
                return acc + durall_v[cid * 8 + m, pl.ds(q * LANES, LANES)]
            acc = lax.fori_loop(0, T // LANES, acc_body,
                                jnp.zeros((LANES,), jnp.int32))
            melvec = jnp.where(iota == m, jnp.sum(acc), melvec)
        mel8_v[...] = melvec
        pltpu.sync_copy(mel8_v.at[pl.ds(0, 8)], mel_hbm.at[pl.ds(cid * 8, 8)])

    vend = jnp.clip(total - base, 0, POS_PER_W)
    zf = jnp.zeros((LANES,), jnp.float32)

    @pl.when(vend <= POS_PER_W - CHUNK)
    def _():
        def zb_body(j, carry):
            for q in range(D // LANES):
                zb_v[j, pl.ds(q * LANES, LANES)] = zf
            return carry
        lax.fori_loop(0, CHUNK, zb_body, 0)
    for i in range(NCHUNK):
        k = i % 2
        pend[k].wait()
        v_i = jnp.clip(vend - i * CHUNK, 0, CHUNK)
        dst = out_hbm.at[b, pl.ds(base + i * CHUNK, CHUNK)]

        @pl.when(v_i == CHUNK)
        def _(k=k, dst=dst):
            pltpu.sync_copy(rows_v.at[k], dst)

        @pl.when(jnp.logical_and(v_i > 0, v_i < CHUNK))
        def _(k=k, dst=dst, v_i=v_i):
            def row_zero(j, carry):
                for q in range(D // LANES):
                    rows_v[k, j, pl.ds(q * LANES, LANES)] = zf
                return carry
            lax.fori_loop(v_i, CHUNK, row_zero, 0)
            pltpu.sync_copy(rows_v.at[k], dst)

        @pl.when(v_i == 0)
        def _(dst=dst):
            pltpu.sync_copy(zb_v, dst)

        if i + 2 < NCHUNK:
            pend[k] = gstart(i + 2, k)


@jax.jit
def _regulate(tbl, duration):
    mesh = plsc.VectorSubcoreMesh(core_axis_name="c", subcore_axis_name="s")
    fn = pl.kernel(
        _body,
        out_type=(jax.ShapeDtypeStruct((B, L, D), jnp.float32),
                  jax.ShapeDtypeStruct((B,), jnp.int32)),
        mesh=mesh,
        compiler_params=pltpu.CompilerParams(needs_layout_passes=False),
        scratch_types=[
            pltpu.VMEM((L,), jnp.int32),
            pltpu.VMEM((T,), jnp.int32),
            pltpu.VMEM((B, T), jnp.int32),
            pltpu.VMEM((LANES,), jnp.int32),
            pltpu.VMEM((2, CHUNK, D), jnp.float32),
            pltpu.VMEM((CHUNK, D), jnp.float32),
            pltpu.SemaphoreType.DMA,
            pltpu.SemaphoreType.DMA,
        ],
    )
    return fn(tbl, duration)


def kernel(x, duration, max_len):
    del max_len
    return _regulate(x.reshape(B * T, D), duration.astype(jnp.int32))
